# Initial kernel scaffold; baseline (speedup 1.0000x reference)
import jax
import jax.numpy as jnp
from jax.experimental import pallas as pl


def kernel(logits, labels):
    raise NotImplementedError("write your pallas kernel here")


# speedup vs baseline: 2.7988x; 2.7988x over previous
import jax
import jax.numpy as jnp
from jax.experimental import pallas as pl
from jax.experimental.pallas import tpu as pltpu

_R = 2048
_L = 128
_LOGP = 18


def _loss_kernel(logits_ref, labels_ref, out_ref):
    x = logits_ref[0]
    lab = labels_ref[0]
    labf = lab.astype(jnp.float32)
    e = 1.0 - x * (2.0 * labf - 1.0)

    bits = pltpu.bitcast(e, jnp.int32)
    key = jnp.where(bits >= 0, bits, bits ^ jnp.int32(0x7FFFFFFF))
    key = (key & jnp.int32(-2)) | lab

    rows_i = jax.lax.broadcasted_iota(jnp.int32, (_R, _L), 0)
    lanes_i = jax.lax.broadcasted_iota(jnp.int32, (_R, _L), 1)
    idx = rows_i * _L + lanes_i

    for ke in range(1, _LOGP + 1):
        for jl in range(ke - 1, -1, -1):
            d = 1 << jl
            bit_u = ((idx >> jl) & 1) == 1
            bit_k = ((idx >> ke) & 1) == 1
            want_max = bit_u == bit_k
            if d >= _L:
                sh = d // _L
                p_up = pltpu.roll(key, _R - sh, 0)
                p_dn = pltpu.roll(key, sh, 0)
            else:
                p_up = pltpu.roll(key, _L - d, 1)
                p_dn = pltpu.roll(key, d, 1)
            partner = jnp.where(bit_u, p_dn, p_up)
            key = jnp.where(want_max, jnp.maximum(key, partner),
                            jnp.minimum(key, partner))

    gt = (key & 1).astype(jnp.float32)
    ebits = jnp.where(key >= 0, key, key ^ jnp.int32(0x7FFFFFFF))
    relu_e = jnp.maximum(pltpu.bitcast(ebits, jnp.float32), 0.0)

    ut = (jax.lax.broadcasted_iota(jnp.int32, (_L, _L), 0)
          <= jax.lax.broadcasted_iota(jnp.int32, (_L, _L), 1)).astype(jnp.float32)
    c1 = jax.lax.dot(gt, ut, precision=jax.lax.Precision.HIGHEST)
    t = c1[:, _L - 1:_L]
    p = jnp.concatenate([jnp.zeros((1, 1), jnp.float32), t[:-1]], axis=0)
    sh = 1
    while sh < _R:
        p = p + jnp.concatenate(
            [jnp.zeros((sh, 1), jnp.float32), p[:-sh]], axis=0)
        sh *= 2
    cum_gt = c1 + p

    g_total = jnp.sum(gt)
    pos = (idx + 1).astype(jnp.float32)
    inter = g_total - cum_gt
    union = g_total + pos - cum_gt
    jacc = 1.0 - inter / union

    a = pltpu.roll(jacc, 1, 1)
    arow = jnp.concatenate([jnp.zeros((1, _L), jnp.float32), a[:-1]], axis=0)
    prev = jnp.where(lanes_i == 0, arow, a)
    out_ref[0] = jnp.full((8, 128), jnp.sum(relu_e * (jacc - prev)),
                          dtype=jnp.float32)


def kernel(logits, labels):
    b = logits.shape[0]
    lf = logits.reshape(b, _R, _L)
    lb = labels.reshape(b, _R, _L)
    losses = pl.pallas_call(
        _loss_kernel,
        grid=(b,),
        in_specs=[
            pl.BlockSpec((1, _R, _L), lambda i: (i, 0, 0)),
            pl.BlockSpec((1, _R, _L), lambda i: (i, 0, 0)),
        ],
        out_specs=pl.BlockSpec((1, 8, 128), lambda i: (i, 0, 0)),
        out_shape=jax.ShapeDtypeStruct((b, 8, 128), jnp.float32),
    )(lf, lb)
    return jnp.mean(losses[:, 0, 0])


# speedup vs baseline: 3.6193x; 1.2932x over previous
import jax
import jax.numpy as jnp
from jax.experimental import pallas as pl
from jax.experimental.pallas import tpu as pltpu

_R = 2048
_L = 128
_LOGR = 11
_LOGP = 18


def _loss_kernel(logits_ref, labels_ref, out_ref):
    x = logits_ref[0]
    lab = labels_ref[0]
    labf = lab.astype(jnp.float32)
    e = 1.0 - x * (2.0 * labf - 1.0)

    bits = pltpu.bitcast(e, jnp.int32)
    key = jnp.where(bits >= 0, bits, bits ^ jnp.int32(0x7FFFFFFF))
    key = (key & jnp.int32(-2)) | lab

    rows_i = jax.lax.broadcasted_iota(jnp.int32, (_R, _L), 0)
    lanes_i = jax.lax.broadcasted_iota(jnp.int32, (_R, _L), 1)

    def flat_bit(b):
        if b < _LOGR:
            return ((rows_i >> b) & 1) == 1
        return ((lanes_i >> (b - _LOGR)) & 1) == 1

    for ke in range(1, _LOGP + 1):
        for jl in range(ke - 1, -1, -1):
            if jl >= _LOGR:
                d = 1 << (jl - _LOGR)
                bit_u = flat_bit(jl)
                want_max = bit_u == flat_bit(ke)
                partner = jnp.where(bit_u, pltpu.roll(key, d, 1),
                                    pltpu.roll(key, _L - d, 1))
                key = jnp.where(want_max, jnp.maximum(key, partner),
                                jnp.minimum(key, partner))
            else:
                m = 1 << jl
                if m >= 8:
                    o = _R // (2 * m)
                    v = key.reshape(o, 2, m, _L)
                    a, b = v[:, 0], v[:, 1]
                    lo = jnp.minimum(a, b)
                    hi = jnp.maximum(a, b)
                    if ke < _LOGR:
                        o_i = jax.lax.broadcasted_iota(jnp.int32, (o, 1, 1), 0)
                        dmask = ((o_i >> (ke - jl - 1)) & 1) == 1
                    else:
                        l_i = jax.lax.broadcasted_iota(jnp.int32, (1, 1, _L), 2)
                        dmask = ((l_i >> (ke - _LOGR)) & 1) == 1
                    newa = jnp.where(dmask, lo, hi)
                    newb = jnp.where(dmask, hi, lo)
                    key = jnp.concatenate(
                        [newa[:, None], newb[:, None]], axis=1
                    ).reshape(_R, _L)
                else:
                    bit_u = flat_bit(jl)
                    want_max = bit_u == flat_bit(ke)
                    partner = jnp.where(bit_u, pltpu.roll(key, m, 0),
                                        pltpu.roll(key, _R - m, 0))
                    key = jnp.where(want_max, jnp.maximum(key, partner),
                                    jnp.minimum(key, partner))

    gt = (key & 1).astype(jnp.float32)
    ebits = jnp.where(key >= 0, key, key ^ jnp.int32(0x7FFFFFFF))
    relu_e = jnp.maximum(pltpu.bitcast(ebits, jnp.float32), 0.0)

    c = gt
    sh = 1
    while sh < _R:
        c = c + jnp.concatenate(
            [jnp.zeros((sh, _L), jnp.float32), c[:-sh]], axis=0)
        sh *= 2
    t = c[_R - 1:_R, :]
    lts = (jax.lax.broadcasted_iota(jnp.int32, (_L, _L), 0)
           < jax.lax.broadcasted_iota(jnp.int32, (_L, _L), 1)).astype(jnp.float32)
    colprefix = jax.lax.dot(t, lts, precision=jax.lax.Precision.HIGHEST)
    cum_gt = c + colprefix

    g_total = jnp.sum(gt)
    pos = (lanes_i * _R + rows_i + 1).astype(jnp.float32)
    inter = g_total - cum_gt
    union = g_total + pos - cum_gt
    jacc = 1.0 - inter / union

    a = pltpu.roll(jacc, 1, 0)
    prev = jnp.where(rows_i == 0, pltpu.roll(a, 1, 1), a)
    prev = jnp.where((rows_i == 0) & (lanes_i == 0), 0.0, prev)
    out_ref[0] = jnp.full((8, 128), jnp.sum(relu_e * (jacc - prev)),
                          dtype=jnp.float32)


def kernel(logits, labels):
    b = logits.shape[0]
    lf = logits.reshape(b, _R, _L)
    lb = labels.reshape(b, _R, _L)
    losses = pl.pallas_call(
        _loss_kernel,
        grid=(b,),
        in_specs=[
            pl.BlockSpec((1, _R, _L), lambda i: (i, 0, 0)),
            pl.BlockSpec((1, _R, _L), lambda i: (i, 0, 0)),
        ],
        out_specs=pl.BlockSpec((1, 8, 128), lambda i: (i, 0, 0)),
        out_shape=jax.ShapeDtypeStruct((b, 8, 128), jnp.float32),
    )(lf, lb)
    return jnp.mean(losses[:, 0, 0])


# speedup vs baseline: 4.5672x; 1.2619x over previous
import jax
import jax.numpy as jnp
from jax.experimental import pallas as pl
from jax.experimental.pallas import tpu as pltpu

_R = 2048
_L = 128
_B = 8
_LOGR = 11
_LOGP = 18


def _loss_kernel(logits_ref, labels_ref, out_ref):
    x = logits_ref[...]
    lab = labels_ref[...]
    labf = lab.astype(jnp.float32)
    e = 1.0 - x * (2.0 * labf - 1.0)

    bits = pltpu.bitcast(e, jnp.int32)
    key = jnp.where(bits >= 0, bits, bits ^ jnp.int32(0x7FFFFFFF))
    key = (key & jnp.int32(-2)) | lab

    shp = (_B, _R, _L)
    rows_i = jax.lax.broadcasted_iota(jnp.int32, shp, 1)
    lanes_i = jax.lax.broadcasted_iota(jnp.int32, shp, 2)

    def flat_bit(b):
        if b < _LOGR:
            return ((rows_i >> b) & 1) == 1
        return ((lanes_i >> (b - _LOGR)) & 1) == 1

    for ke in range(1, _LOGP + 1):
        for jl in range(ke - 1, -1, -1):
            if jl >= _LOGR:
                d = 1 << (jl - _LOGR)
                bit_u = flat_bit(jl)
                want_max = bit_u == flat_bit(ke)
                partner = jnp.where(bit_u, pltpu.roll(key, d, 2),
                                    pltpu.roll(key, _L - d, 2))
                key = jnp.where(want_max, jnp.maximum(key, partner),
                                jnp.minimum(key, partner))
            else:
                m = 1 << jl
                if m >= 8:
                    o = _R // (2 * m)
                    v = key.reshape(_B, o, 2, m, _L)
                    a, b = v[:, :, 0], v[:, :, 1]
                    lo = jnp.minimum(a, b)
                    hi = jnp.maximum(a, b)
                    if ke < _LOGR:
                        o_i = jax.lax.broadcasted_iota(
                            jnp.int32, (1, o, 1, 1), 1)
                        dmask = ((o_i >> (ke - jl - 1)) & 1) == 1
                    else:
                        l_i = jax.lax.broadcasted_iota(
                            jnp.int32, (1, 1, 1, _L), 3)
                        dmask = ((l_i >> (ke - _LOGR)) & 1) == 1
                    newa = jnp.where(dmask, lo, hi)
                    newb = jnp.where(dmask, hi, lo)
                    key = jnp.concatenate(
                        [newa[:, :, None], newb[:, :, None]], axis=2
                    ).reshape(_B, _R, _L)
                else:
                    bit_u = flat_bit(jl)
                    want_max = bit_u == flat_bit(ke)
                    partner = jnp.where(bit_u, pltpu.roll(key, m, 1),
                                        pltpu.roll(key, _R - m, 1))
                    key = jnp.where(want_max, jnp.maximum(key, partner),
                                    jnp.minimum(key, partner))

    gt = (key & 1).astype(jnp.float32)
    ebits = jnp.where(key >= 0, key, key ^ jnp.int32(0x7FFFFFFF))
    relu_e = jnp.maximum(pltpu.bitcast(ebits, jnp.float32), 0.0)

    c = gt
    sh = 1
    while sh < _R:
        c = c + jnp.concatenate(
            [jnp.zeros((_B, sh, _L), jnp.float32), c[:, :-sh]], axis=1)
        sh *= 2
    t = c[:, _R - 1, :]
    ii = jax.lax.broadcasted_iota(jnp.int32, (_L, _L), 0)
    jj = jax.lax.broadcasted_iota(jnp.int32, (_L, _L), 1)
    lts = (ii < jj).astype(jnp.float32)
    colprefix = jax.lax.dot(t, lts,
                            precision=jax.lax.Precision.HIGHEST)
    cum_gt = c + colprefix[:, None, :]
    g_total = jnp.sum(t, axis=1)[:, None, None]

    pos = (lanes_i * _R + rows_i + 1).astype(jnp.float32)
    inter = g_total - cum_gt
    union = g_total + pos - cum_gt
    jacc = 1.0 - inter / union

    a = pltpu.roll(jacc, 1, 1)
    prev = jnp.where(rows_i == 0, pltpu.roll(a, 1, 2), a)
    prev = jnp.where((rows_i == 0) & (lanes_i == 0), 0.0, prev)
    contrib = relu_e * (jacc - prev)
    lane_part = jnp.sum(contrib, axis=1)
    ones_l = jnp.ones((_L, _L), jnp.float32)
    img_tot = jax.lax.dot(lane_part, ones_l,
                          precision=jax.lax.Precision.HIGHEST)
    out_ref[...] = img_tot


def kernel(logits, labels):
    b = logits.shape[0]
    lf = logits.reshape(b, _R, _L)
    lb = labels.reshape(b, _R, _L)
    losses = pl.pallas_call(
        _loss_kernel,
        in_specs=[
            pl.BlockSpec((_B, _R, _L), lambda: (0, 0, 0)),
            pl.BlockSpec((_B, _R, _L), lambda: (0, 0, 0)),
        ],
        out_specs=pl.BlockSpec((_B, _L), lambda: (0, 0)),
        out_shape=jax.ShapeDtypeStruct((_B, _L), jnp.float32),
    )(lf, lb)
    return jnp.mean(losses[:, 0])


# speedup vs baseline: 6.7623x; 1.4806x over previous
import jax
import jax.numpy as jnp
from jax.experimental import pallas as pl
from jax.experimental.pallas import tpu as pltpu

_B = 8
_Q = 256
_S = 8
_L = 128
_G = _B * _Q
_LOGP = 18


def _loss_kernel(logits_ref, labels_ref, out_ref):
    x = logits_ref[...]
    lab = labels_ref[...]
    labf = lab.astype(jnp.float32)
    e = 1.0 - x * (2.0 * labf - 1.0)

    bits = pltpu.bitcast(e, jnp.int32)
    key = jnp.where(bits >= 0, bits, bits ^ jnp.int32(0x7FFFFFFF))
    key = (key & jnp.int32(-2)) | lab

    s_col = jax.lax.broadcasted_iota(jnp.int32, (1, _S, 1), 1)
    l_row = jax.lax.broadcasted_iota(jnp.int32, (1, 1, _L), 2)

    def hi_bit(b):
        if b < 11:
            return ((s_col >> (b - 8)) & 1) == 1
        return ((l_row >> (b - 11)) & 1) == 1

    for ke in range(1, _LOGP + 1):
        for jl in range(ke - 1, -1, -1):
            if jl < 8:
                m = 1 << jl
                o = _G // (2 * m)
                v = key.reshape(o, 2, m, _S, _L)
                a, b = v[:, 0], v[:, 1]
                lo = jnp.minimum(a, b)
                hi = jnp.maximum(a, b)
                if ke < 8:
                    oi = jax.lax.broadcasted_iota(jnp.int32, (o, 1, 1, 1), 0)
                    dmask = ((oi >> (ke - jl - 1)) & 1) == 1
                elif ke < 11:
                    si = jax.lax.broadcasted_iota(jnp.int32, (1, 1, _S, 1), 2)
                    dmask = ((si >> (ke - 8)) & 1) == 1
                else:
                    li = jax.lax.broadcasted_iota(jnp.int32, (1, 1, 1, _L), 3)
                    dmask = ((li >> (ke - 11)) & 1) == 1
                newa = jnp.where(dmask, lo, hi)
                newb = jnp.where(dmask, hi, lo)
                key = jnp.concatenate(
                    [newa[:, None], newb[:, None]], axis=1
                ).reshape(_G, _S, _L)
            elif jl < 11:
                d = 1 << (jl - 8)
                bit_u = hi_bit(jl)
                want_max = bit_u == hi_bit(ke)
                partner = jnp.where(bit_u, pltpu.roll(key, d, 1),
                                    pltpu.roll(key, _S - d, 1))
                key = jnp.where(want_max, jnp.maximum(key, partner),
                                jnp.minimum(key, partner))
            else:
                d = 1 << (jl - 11)
                bit_u = hi_bit(jl)
                want_max = bit_u == hi_bit(ke)
                partner = jnp.where(bit_u, pltpu.roll(key, d, 2),
                                    pltpu.roll(key, _L - d, 2))
                key = jnp.where(want_max, jnp.maximum(key, partner),
                                jnp.minimum(key, partner))

    gt = (key & 1).astype(jnp.float32)
    ebits = jnp.where(key >= 0, key, key ^ jnp.int32(0x7FFFFFFF))
    relu_e = jnp.maximum(pltpu.bitcast(ebits, jnp.float32), 0.0)

    c = gt.reshape(_B, _Q, _S, _L)
    sh = 1
    while sh < _Q:
        c = c + jnp.concatenate(
            [jnp.zeros((_B, sh, _S, _L), jnp.float32), c[:, :-sh]], axis=1)
        sh *= 2
    t = c[:, _Q - 1]
    p = jnp.concatenate(
        [jnp.zeros((_B, 1, _L), jnp.float32), t[:, :-1]], axis=1)
    sh = 1
    while sh < _S:
        p = p + jnp.concatenate(
            [jnp.zeros((_B, sh, _L), jnp.float32), p[:, :-sh]], axis=1)
        sh *= 2
    colt = jnp.sum(t, axis=1)
    ii = jax.lax.broadcasted_iota(jnp.int32, (_L, _L), 0)
    jj = jax.lax.broadcasted_iota(jnp.int32, (_L, _L), 1)
    lts = (ii < jj).astype(jnp.float32)
    colprefix = jax.lax.dot(colt, lts,
                            precision=jax.lax.Precision.HIGHEST)
    cum_gt = c + p[:, None] + colprefix[:, None, None, :]
    g_total = jnp.sum(colt, axis=1)[:, None, None, None]

    q_i = jax.lax.broadcasted_iota(jnp.int32, (1, _Q, 1, 1), 1)
    s_i = jax.lax.broadcasted_iota(jnp.int32, (1, 1, _S, 1), 2)
    l_i = jax.lax.broadcasted_iota(jnp.int32, (1, 1, 1, _L), 3)
    pos = (l_i * (_Q * _S) + s_i * _Q + q_i + 1).astype(jnp.float32)
    inter = g_total - cum_gt
    union = g_total + pos - cum_gt
    jacc = 1.0 - inter / union

    a = jnp.concatenate([jacc[:, _Q - 1:], jacc[:, :-1]], axis=1)
    b2 = pltpu.roll(a, 1, 2)
    c2 = pltpu.roll(b2, 1, 3)
    prev = jnp.where(q_i == 0, jnp.where(s_i == 0, c2, b2), a)
    prev = jnp.where((q_i == 0) & (s_i == 0) & (l_i == 0), 0.0, prev)
    contrib = relu_e.reshape(_B, _Q, _S, _L) * (jacc - prev)
    part = jnp.sum(jnp.sum(contrib, axis=1), axis=1)
    ones_l = jnp.ones((_L, _L), jnp.float32)
    img_tot = jax.lax.dot(part, ones_l,
                          precision=jax.lax.Precision.HIGHEST)
    out_ref[...] = img_tot


def kernel(logits, labels):
    lf = logits.reshape(_G, _S, _L)
    lb = labels.reshape(_G, _S, _L)
    losses = pl.pallas_call(
        _loss_kernel,
        in_specs=[
            pl.BlockSpec((_G, _S, _L), lambda: (0, 0, 0)),
            pl.BlockSpec((_G, _S, _L), lambda: (0, 0, 0)),
        ],
        out_specs=pl.BlockSpec((_B, _L), lambda: (0, 0)),
        out_shape=jax.ShapeDtypeStruct((_B, _L), jnp.float32),
    )(lf, lb)
    return jnp.mean(losses[:, 0])


# speedup vs baseline: 7.3529x; 1.0873x over previous
import jax
import jax.numpy as jnp
from jax.experimental import pallas as pl
from jax.experimental.pallas import tpu as pltpu

_G = 2048
_S = 8
_L = 128
_LOGP = 18


def _loss_kernel(logits_ref, labels_ref, out_ref):
    x = logits_ref[...]
    lab = labels_ref[...]
    labf = lab.astype(jnp.float32)
    e = 1.0 - x * (2.0 * labf - 1.0)

    bits = pltpu.bitcast(e, jnp.int32)
    key = jnp.where(bits >= 0, bits, bits ^ jnp.int32(0x7FFFFFFF))
    key = (key & jnp.int32(-2)) | lab

    l_i = jax.lax.broadcasted_iota(jnp.int32, (1, 1, _L), 2)

    def lane_bit(b):
        return ((l_i >> (b - 11)) & 1) == 1

    for ke in range(1, _LOGP + 1):
        for jl in range(ke - 1, -1, -1):
            if jl < 11:
                m = 1 << jl
                if ke <= 10:
                    q = 1 << (ke - jl - 1)
                    o2 = _G // (4 * m * q)
                    v = key.reshape(o2, 2, q, 2, m, _S, _L)
                    a, b = v[:, :, :, 0], v[:, :, :, 1]
                    lo = jnp.minimum(a, b)
                    hi = jnp.maximum(a, b)
                    newa = jnp.concatenate([lo[:, :1], hi[:, 1:]], axis=1)
                    newb = jnp.concatenate([hi[:, :1], lo[:, 1:]], axis=1)
                    key = jnp.concatenate(
                        [newa[:, :, :, None], newb[:, :, :, None]], axis=3
                    ).reshape(_G, _S, _L)
                else:
                    o = _G // (2 * m)
                    v = key.reshape(o, 2, m, _S, _L)
                    a, b = v[:, 0], v[:, 1]
                    lo = jnp.minimum(a, b)
                    hi = jnp.maximum(a, b)
                    li = jax.lax.broadcasted_iota(jnp.int32, (1, 1, 1, _L), 3)
                    dmask = ((li >> (ke - 11)) & 1) == 1
                    newa = jnp.where(dmask, lo, hi)
                    newb = jnp.where(dmask, hi, lo)
                    key = jnp.concatenate(
                        [newa[:, None], newb[:, None]], axis=1
                    ).reshape(_G, _S, _L)
            else:
                d = 1 << (jl - 11)
                bit_u = lane_bit(jl)
                want_max = bit_u == lane_bit(ke)
                partner = jnp.where(bit_u, pltpu.roll(key, d, 2),
                                    pltpu.roll(key, _L - d, 2))
                key = jnp.where(want_max, jnp.maximum(key, partner),
                                jnp.minimum(key, partner))

    gt = (key & 1).astype(jnp.float32)
    ebits = jnp.where(key >= 0, key, key ^ jnp.int32(0x7FFFFFFF))
    relu_e = jnp.maximum(pltpu.bitcast(ebits, jnp.float32), 0.0)

    c = gt
    sh = 1
    while sh < _G:
        c = c + jnp.concatenate(
            [jnp.zeros((sh, _S, _L), jnp.float32), c[:-sh]], axis=0)
        sh *= 2
    t = c[_G - 1]
    ii = jax.lax.broadcasted_iota(jnp.int32, (_L, _L), 0)
    jj = jax.lax.broadcasted_iota(jnp.int32, (_L, _L), 1)
    lts = (ii < jj).astype(jnp.float32)
    p1 = jax.lax.dot(t, lts, precision=jax.lax.Precision.HIGHEST)
    cum_gt = c + p1[None]
    g_total = jnp.sum(t, axis=1)[None, :, None]

    g_i = jax.lax.broadcasted_iota(jnp.int32, (_G, 1, 1), 0)
    pos = (g_i + l_i * _G + 1).astype(jnp.float32)
    inter = g_total - cum_gt
    union = g_total + pos - cum_gt
    jacc = 1.0 - inter / union

    a = jnp.concatenate([jacc[_G - 1:], jacc[:-1]], axis=0)
    b2 = pltpu.roll(a, 1, 2)
    prev = jnp.where(g_i == 0, b2, a)
    prev = jnp.where((g_i == 0) & (l_i == 0), 0.0, prev)
    contrib = relu_e * (jacc - prev)
    part = jnp.sum(contrib, axis=0)
    ones_l = jnp.ones((_L, _L), jnp.float32)
    img_tot = jax.lax.dot(part, ones_l,
                          precision=jax.lax.Precision.HIGHEST)
    out_ref[...] = img_tot


def kernel(logits, labels):
    lf = jnp.transpose(logits.reshape(_S, _G, _L), (1, 0, 2))
    lb = jnp.transpose(labels.reshape(_S, _G, _L), (1, 0, 2))
    losses = pl.pallas_call(
        _loss_kernel,
        in_specs=[
            pl.BlockSpec((_G, _S, _L), lambda: (0, 0, 0)),
            pl.BlockSpec((_G, _S, _L), lambda: (0, 0, 0)),
        ],
        out_specs=pl.BlockSpec((_S, _L), lambda: (0, 0)),
        out_shape=jax.ShapeDtypeStruct((_S, _L), jnp.float32),
    )(lf, lb)
    return jnp.mean(losses[:, 0])


# speedup vs baseline: 8.3334x; 1.1334x over previous
import jax
import jax.numpy as jnp
from jax.experimental import pallas as pl
from jax.experimental.pallas import tpu as pltpu

_G = 2048
_S = 8
_L = 128
_LOGP = 18


def _loss_kernel(logits_ref, labels_ref, out_ref):
    x = logits_ref[...]
    lab = labels_ref[...]
    labf = lab.astype(jnp.float32)
    e = 1.0 - x * (2.0 * labf - 1.0)

    bits = pltpu.bitcast(e, jnp.int32)
    key = jnp.where(bits >= 0, bits, bits ^ jnp.int32(0x7FFFFFFF))
    key = (key & jnp.int32(-2)) | lab

    l_i = jax.lax.broadcasted_iota(jnp.int32, (1, 1, _L), 2)
    g_i0 = jax.lax.broadcasted_iota(jnp.int32, (_G, 1, 1), 0)

    def lane_bit(b):
        return ((l_i >> (b - 11)) & 1) == 1

    def flip_mask(ke):
        if ke <= 10:
            return ((g_i0 >> ke) & 1) - jnp.int32(1)
        if ke <= 17:
            return ((l_i >> (ke - 11)) & 1) - jnp.int32(1)
        return jnp.int32(-1)

    key = key ^ flip_mask(1)
    for ke in range(1, _LOGP + 1):
        for jl in range(ke - 1, -1, -1):
            if jl < 11:
                m = 1 << jl
                o = _G // (2 * m)
                v = key.reshape(o, 2, m, _S, _L)
                a, b = v[:, 0], v[:, 1]
                lo = jnp.minimum(a, b)
                hi = jnp.maximum(a, b)
                key = jnp.concatenate(
                    [lo[:, None], hi[:, None]], axis=1).reshape(_G, _S, _L)
            else:
                d = 1 << (jl - 11)
                bit_u = lane_bit(jl)
                y = pltpu.roll(key, _L - d, 2)
                lo = jnp.minimum(key, y)
                hi = jnp.maximum(key, y)
                key = jnp.where(bit_u, pltpu.roll(hi, d, 2), lo)
        if ke < _LOGP:
            key = key ^ (flip_mask(ke) ^ flip_mask(ke + 1))
        else:
            key = key ^ flip_mask(ke)

    gt = (key & 1).astype(jnp.float32)
    ebits = jnp.where(key >= 0, key, key ^ jnp.int32(0x7FFFFFFF))
    relu_e = jnp.maximum(pltpu.bitcast(ebits, jnp.float32), 0.0)

    c = gt
    sh = 1
    while sh < _G:
        c = c + jnp.concatenate(
            [jnp.zeros((sh, _S, _L), jnp.float32), c[:-sh]], axis=0)
        sh *= 2
    t = c[_G - 1]
    ii = jax.lax.broadcasted_iota(jnp.int32, (_L, _L), 0)
    jj = jax.lax.broadcasted_iota(jnp.int32, (_L, _L), 1)
    lts = (ii < jj).astype(jnp.float32)
    p1 = jax.lax.dot(t, lts, precision=jax.lax.Precision.HIGHEST)
    cum_gt = c + p1[None]
    g_total = jnp.sum(t, axis=1)[None, :, None]

    g_i = jax.lax.broadcasted_iota(jnp.int32, (_G, 1, 1), 0)
    pos = (g_i + l_i * _G + 1).astype(jnp.float32)
    inter = g_total - cum_gt
    union = g_total + pos - cum_gt
    jacc = 1.0 - inter / union

    a = jnp.concatenate([jacc[_G - 1:], jacc[:-1]], axis=0)
    b2 = pltpu.roll(a, 1, 2)
    prev = jnp.where(g_i == 0, b2, a)
    prev = jnp.where((g_i == 0) & (l_i == 0), 0.0, prev)
    contrib = relu_e * (jacc - prev)
    part = jnp.sum(contrib, axis=0)
    ones_l = jnp.ones((_L, _L), jnp.float32)
    img_tot = jax.lax.dot(part, ones_l,
                          precision=jax.lax.Precision.HIGHEST)
    out_ref[...] = img_tot


def kernel(logits, labels):
    lf = jnp.transpose(logits.reshape(_S, _G, _L), (1, 0, 2))
    lb = jnp.transpose(labels.reshape(_S, _G, _L), (1, 0, 2))
    losses = pl.pallas_call(
        _loss_kernel,
        in_specs=[
            pl.BlockSpec((_G, _S, _L), lambda: (0, 0, 0)),
            pl.BlockSpec((_G, _S, _L), lambda: (0, 0, 0)),
        ],
        out_specs=pl.BlockSpec((_S, _L), lambda: (0, 0)),
        out_shape=jax.ShapeDtypeStruct((_S, _L), jnp.float32),
    )(lf, lb)
    return jnp.mean(losses[:, 0])


# speedup vs baseline: 8.3595x; 1.0031x over previous
import jax
import jax.numpy as jnp
from jax.experimental import pallas as pl
from jax.experimental.pallas import tpu as pltpu

_G = 2048
_S = 8
_L = 128
_LOGP = 18


def _loss_kernel(logits_ref, labels_ref, out_ref):
    x = logits_ref[...]
    lab = labels_ref[...]
    labf = lab.astype(jnp.float32)
    e = 1.0 - x * (2.0 * labf - 1.0)

    bits = pltpu.bitcast(e, jnp.int32)
    key = jnp.where(bits >= 0, bits, bits ^ jnp.int32(0x7FFFFFFF))
    key = (key & jnp.int32(-2)) | lab

    l_i = jax.lax.broadcasted_iota(jnp.int32, (1, 1, _L), 2)
    g_i0 = jax.lax.broadcasted_iota(jnp.int32, (_G, 1, 1), 0)

    def lane_bit(b):
        return ((l_i >> (b - 11)) & 1) == 1

    def flip_mask(ke):
        if ke <= 10:
            return ((g_i0 >> ke) & 1) - jnp.int32(1)
        if ke <= 17:
            return ((l_i >> (ke - 11)) & 1) - jnp.int32(1)
        return jnp.int32(-1)

    def low_merge(key, kbits):
        n = 1 << kbits
        w = key.reshape(_G // n, n, _S, _L)
        parts = [w[:, i] for i in range(n)]
        for jl in range(kbits - 1, -1, -1):
            d = 1 << jl
            for i in range(n):
                if not (i & d):
                    a, b = parts[i], parts[i + d]
                    parts[i] = jnp.minimum(a, b)
                    parts[i + d] = jnp.maximum(a, b)
        return jnp.concatenate(
            [p[:, None] for p in parts], axis=1).reshape(_G, _S, _L)

    _LOW = 4
    key = key ^ flip_mask(1)
    for ke in range(1, _LOGP + 1):
        for jl in range(ke - 1, min(ke, _LOW) - 1, -1):
            if jl < 11:
                m = 1 << jl
                o = _G // (2 * m)
                v = key.reshape(o, 2, m, _S, _L)
                a, b = v[:, 0], v[:, 1]
                lo = jnp.minimum(a, b)
                hi = jnp.maximum(a, b)
                key = jnp.concatenate(
                    [lo[:, None], hi[:, None]], axis=1).reshape(_G, _S, _L)
            else:
                d = 1 << (jl - 11)
                bit_u = lane_bit(jl)
                y = pltpu.roll(key, _L - d, 2)
                lo = jnp.minimum(key, y)
                hi = jnp.maximum(key, y)
                key = jnp.where(bit_u, pltpu.roll(hi, d, 2), lo)
        key = low_merge(key, min(ke, _LOW))
        if ke < _LOGP:
            key = key ^ (flip_mask(ke) ^ flip_mask(ke + 1))
        else:
            key = key ^ flip_mask(ke)

    gt = (key & 1).astype(jnp.float32)
    ebits = jnp.where(key >= 0, key, key ^ jnp.int32(0x7FFFFFFF))
    relu_e = jnp.maximum(pltpu.bitcast(ebits, jnp.float32), 0.0)

    c = gt
    sh = 1
    while sh < _G:
        c = c + jnp.concatenate(
            [jnp.zeros((sh, _S, _L), jnp.float32), c[:-sh]], axis=0)
        sh *= 2
    t = c[_G - 1]
    ii = jax.lax.broadcasted_iota(jnp.int32, (_L, _L), 0)
    jj = jax.lax.broadcasted_iota(jnp.int32, (_L, _L), 1)
    lts = (ii < jj).astype(jnp.float32)
    p1 = jax.lax.dot(t, lts, precision=jax.lax.Precision.HIGHEST)
    cum_gt = c + p1[None]
    g_total = jnp.sum(t, axis=1)[None, :, None]

    g_i = jax.lax.broadcasted_iota(jnp.int32, (_G, 1, 1), 0)
    pos = (g_i + l_i * _G + 1).astype(jnp.float32)
    inter = g_total - cum_gt
    union = g_total + pos - cum_gt
    jacc = 1.0 - inter / union

    a = jnp.concatenate([jacc[_G - 1:], jacc[:-1]], axis=0)
    b2 = pltpu.roll(a, 1, 2)
    prev = jnp.where(g_i == 0, b2, a)
    prev = jnp.where((g_i == 0) & (l_i == 0), 0.0, prev)
    contrib = relu_e * (jacc - prev)
    part = jnp.sum(contrib, axis=0)
    ones_l = jnp.ones((_L, _L), jnp.float32)
    img_tot = jax.lax.dot(part, ones_l,
                          precision=jax.lax.Precision.HIGHEST)
    out_ref[...] = img_tot


def kernel(logits, labels):
    lf = jnp.transpose(logits.reshape(_S, _G, _L), (1, 0, 2))
    lb = jnp.transpose(labels.reshape(_S, _G, _L), (1, 0, 2))
    losses = pl.pallas_call(
        _loss_kernel,
        in_specs=[
            pl.BlockSpec((_G, _S, _L), lambda: (0, 0, 0)),
            pl.BlockSpec((_G, _S, _L), lambda: (0, 0, 0)),
        ],
        out_specs=pl.BlockSpec((_S, _L), lambda: (0, 0)),
        out_shape=jax.ShapeDtypeStruct((_S, _L), jnp.float32),
    )(lf, lb)
    return jnp.mean(losses[:, 0])


# speedup vs baseline: 8.4351x; 1.0090x over previous
import jax
import jax.numpy as jnp
from jax.experimental import pallas as pl
from jax.experimental.pallas import tpu as pltpu

_G = 2048
_S = 8
_L = 128
_LOGP = 18


def _loss_kernel(logits_ref, labels_ref, out_ref):
    x = logits_ref[...]
    lab = labels_ref[...]
    labf = lab.astype(jnp.float32)
    e = 1.0 - x * (2.0 * labf - 1.0)

    bits = pltpu.bitcast(e, jnp.int32)
    key = jnp.where(bits >= 0, bits, bits ^ jnp.int32(0x7FFFFFFF))
    key = (key & jnp.int32(-2)) | lab

    l_i = jax.lax.broadcasted_iota(jnp.int32, (1, 1, _L), 2)
    g_i0 = jax.lax.broadcasted_iota(jnp.int32, (_G, 1, 1), 0)

    def lane_bit(b):
        return ((l_i >> (b - 11)) & 1) == 1

    def flip_mask(ke):
        if ke <= 10:
            return ((g_i0 >> ke) & 1) - jnp.int32(1)
        if ke <= 17:
            return ((l_i >> (ke - 11)) & 1) - jnp.int32(1)
        return jnp.int32(-1)

    def low_merge(key, kbits):
        n = 1 << kbits
        w = key.reshape(_G // n, n, _S, _L)
        parts = [w[:, i] for i in range(n)]
        for jl in range(kbits - 1, -1, -1):
            d = 1 << jl
            for i in range(n):
                if not (i & d):
                    a, b = parts[i], parts[i + d]
                    parts[i] = jnp.minimum(a, b)
                    parts[i + d] = jnp.maximum(a, b)
        return jnp.concatenate(
            [p[:, None] for p in parts], axis=1).reshape(_G, _S, _L)

    n = 16
    w = key.reshape(_G // n, n, _S, _L)
    parts = [w[:, i] for i in range(n)]
    for ke in range(1, 4):
        for jl in range(ke - 1, -1, -1):
            d = 1 << jl
            for i in range(n):
                if not (i & d):
                    asc = ((i >> ke) & 1) == 1
                    a, b = parts[i], parts[i + d]
                    lo = jnp.minimum(a, b)
                    hi = jnp.maximum(a, b)
                    parts[i], parts[i + d] = (lo, hi) if asc else (hi, lo)
    key = jnp.concatenate(
        [p[:, None] for p in parts], axis=1).reshape(_G, _S, _L)

    _LOW = 5
    key = key ^ flip_mask(4)
    for ke in range(4, _LOGP + 1):
        for jl in range(ke - 1, min(ke, _LOW) - 1, -1):
            if jl < 11:
                m = 1 << jl
                o = _G // (2 * m)
                v = key.reshape(o, 2, m, _S, _L)
                a, b = v[:, 0], v[:, 1]
                lo = jnp.minimum(a, b)
                hi = jnp.maximum(a, b)
                key = jnp.concatenate(
                    [lo[:, None], hi[:, None]], axis=1).reshape(_G, _S, _L)
            else:
                d = 1 << (jl - 11)
                bit_u = lane_bit(jl)
                y = pltpu.roll(key, _L - d, 2)
                lo = jnp.minimum(key, y)
                hi = jnp.maximum(key, y)
                key = jnp.where(bit_u, pltpu.roll(hi, d, 2), lo)
        key = low_merge(key, min(ke, _LOW))
        if ke < _LOGP:
            key = key ^ (flip_mask(ke) ^ flip_mask(ke + 1))
        else:
            key = key ^ flip_mask(ke)

    gt = (key & 1).astype(jnp.float32)
    ebits = jnp.where(key >= 0, key, key ^ jnp.int32(0x7FFFFFFF))
    relu_e = jnp.maximum(pltpu.bitcast(ebits, jnp.float32), 0.0)

    c = gt
    sh = 1
    while sh < _G:
        c = c + jnp.concatenate(
            [jnp.zeros((sh, _S, _L), jnp.float32), c[:-sh]], axis=0)
        sh *= 2
    t = c[_G - 1]
    ii = jax.lax.broadcasted_iota(jnp.int32, (_L, _L), 0)
    jj = jax.lax.broadcasted_iota(jnp.int32, (_L, _L), 1)
    lts = (ii < jj).astype(jnp.float32)
    p1 = jax.lax.dot(t, lts, precision=jax.lax.Precision.HIGHEST)
    cum_gt = c + p1[None]
    g_total = jnp.sum(t, axis=1)[None, :, None]

    g_i = jax.lax.broadcasted_iota(jnp.int32, (_G, 1, 1), 0)
    pos = (g_i + l_i * _G + 1).astype(jnp.float32)
    inter = g_total - cum_gt
    union = g_total + pos - cum_gt
    jacc = 1.0 - inter / union

    a = jnp.concatenate([jacc[_G - 1:], jacc[:-1]], axis=0)
    b2 = pltpu.roll(a, 1, 2)
    prev = jnp.where(g_i == 0, b2, a)
    prev = jnp.where((g_i == 0) & (l_i == 0), 0.0, prev)
    contrib = relu_e * (jacc - prev)
    part = jnp.sum(contrib, axis=0)
    ones_l = jnp.ones((_L, _L), jnp.float32)
    img_tot = jax.lax.dot(part, ones_l,
                          precision=jax.lax.Precision.HIGHEST)
    out_ref[...] = img_tot


def kernel(logits, labels):
    lf = jnp.transpose(logits.reshape(_S, _G, _L), (1, 0, 2))
    lb = jnp.transpose(labels.reshape(_S, _G, _L), (1, 0, 2))
    losses = pl.pallas_call(
        _loss_kernel,
        in_specs=[
            pl.BlockSpec((_G, _S, _L), lambda: (0, 0, 0)),
            pl.BlockSpec((_G, _S, _L), lambda: (0, 0, 0)),
        ],
        out_specs=pl.BlockSpec((_S, _L), lambda: (0, 0)),
        out_shape=jax.ShapeDtypeStruct((_S, _L), jnp.float32),
    )(lf, lb)
    return jnp.mean(losses[:, 0])


# speedup vs baseline: 8.4948x; 1.0071x over previous
import jax
import jax.numpy as jnp
from jax.experimental import pallas as pl
from jax.experimental.pallas import tpu as pltpu

_G = 2048
_S = 8
_L = 128
_LOGP = 18


def _loss_kernel(logits_ref, labels_ref, out_ref):
    x = logits_ref[...]
    lab = labels_ref[...]
    labf = lab.astype(jnp.float32)
    e = 1.0 - x * (2.0 * labf - 1.0)

    bits = pltpu.bitcast(e, jnp.int32)
    key = jnp.where(bits >= 0, bits, bits ^ jnp.int32(0x7FFFFFFF))
    key = (key & jnp.int32(-2)) | lab

    l_i = jax.lax.broadcasted_iota(jnp.int32, (1, 1, _L), 2)
    g_i0 = jax.lax.broadcasted_iota(jnp.int32, (_G, 1, 1), 0)

    def lane_bit(b):
        return ((l_i >> (b - 11)) & 1) == 1

    def flip_mask(ke):
        if ke <= 10:
            return ((g_i0 >> ke) & 1) - jnp.int32(1)
        if ke <= 17:
            return ((l_i >> (ke - 11)) & 1) - jnp.int32(1)
        return jnp.int32(-1)

    def low_merge(key, kbits):
        n = 1 << kbits
        w = key.reshape(_G // n, n, _S, _L)
        parts = [w[:, i] for i in range(n)]
        for jl in range(kbits - 1, -1, -1):
            d = 1 << jl
            for i in range(n):
                if not (i & d):
                    a, b = parts[i], parts[i + d]
                    parts[i] = jnp.minimum(a, b)
                    parts[i + d] = jnp.maximum(a, b)
        return jnp.concatenate(
            [p[:, None] for p in parts], axis=1).reshape(_G, _S, _L)

    n = 16
    w = key.reshape(_G // n, n, _S, _L)
    parts = [w[:, i] for i in range(n)]
    for ke in range(1, 4):
        for jl in range(ke - 1, -1, -1):
            d = 1 << jl
            for i in range(n):
                if not (i & d):
                    asc = ((i >> ke) & 1) == 1
                    a, b = parts[i], parts[i + d]
                    lo = jnp.minimum(a, b)
                    hi = jnp.maximum(a, b)
                    parts[i], parts[i + d] = (lo, hi) if asc else (hi, lo)
    key = jnp.concatenate(
        [p[:, None] for p in parts], axis=1).reshape(_G, _S, _L)

    _LOW = 5
    key = key ^ flip_mask(4)
    for ke in range(4, _LOGP + 1):
        for jl in range(ke - 1, 10, -1):
            d = 1 << (jl - 11)
            bit_u = lane_bit(jl)
            y = pltpu.roll(key, _L - d, 2)
            lo = jnp.minimum(key, y)
            hi = jnp.maximum(key, y)
            key = jnp.where(bit_u, pltpu.roll(hi, d, 2), lo)
        hi_top = min(ke - 1, 10)
        if hi_top >= _LOW:
            w = key.reshape(64, _G // 64, _S, _L)
            hparts = [w[i] for i in range(64)]
            for jl in range(hi_top, _LOW - 1, -1):
                pd = 1 << (jl - _LOW)
                for i in range(64):
                    if not (i & pd):
                        a, b = hparts[i], hparts[i + pd]
                        hparts[i] = jnp.minimum(a, b)
                        hparts[i + pd] = jnp.maximum(a, b)
            key = jnp.concatenate(
                [p[None] for p in hparts], axis=0).reshape(_G, _S, _L)
        key = low_merge(key, min(ke, _LOW))
        if ke < _LOGP:
            key = key ^ (flip_mask(ke) ^ flip_mask(ke + 1))
        else:
            key = key ^ flip_mask(ke)

    gt = (key & 1).astype(jnp.float32)
    ebits = jnp.where(key >= 0, key, key ^ jnp.int32(0x7FFFFFFF))
    relu_e = jnp.maximum(pltpu.bitcast(ebits, jnp.float32), 0.0)

    c = gt
    sh = 1
    while sh < _G:
        c = c + jnp.concatenate(
            [jnp.zeros((sh, _S, _L), jnp.float32), c[:-sh]], axis=0)
        sh *= 2
    t = c[_G - 1]
    ii = jax.lax.broadcasted_iota(jnp.int32, (_L, _L), 0)
    jj = jax.lax.broadcasted_iota(jnp.int32, (_L, _L), 1)
    lts = (ii < jj).astype(jnp.float32)
    p1 = jax.lax.dot(t, lts, precision=jax.lax.Precision.HIGHEST)
    cum_gt = c + p1[None]
    g_total = jnp.sum(t, axis=1)[None, :, None]

    g_i = jax.lax.broadcasted_iota(jnp.int32, (_G, 1, 1), 0)
    pos = (g_i + l_i * _G + 1).astype(jnp.float32)
    inter = g_total - cum_gt
    union = g_total + pos - cum_gt
    jacc = 1.0 - inter / union

    a = jnp.concatenate([jacc[_G - 1:], jacc[:-1]], axis=0)
    b2 = pltpu.roll(a, 1, 2)
    prev = jnp.where(g_i == 0, b2, a)
    prev = jnp.where((g_i == 0) & (l_i == 0), 0.0, prev)
    contrib = relu_e * (jacc - prev)
    part = jnp.sum(contrib, axis=0)
    ones_l = jnp.ones((_L, _L), jnp.float32)
    img_tot = jax.lax.dot(part, ones_l,
                          precision=jax.lax.Precision.HIGHEST)
    out_ref[...] = img_tot


def kernel(logits, labels):
    lf = jnp.transpose(logits.reshape(_S, _G, _L), (1, 0, 2))
    lb = jnp.transpose(labels.reshape(_S, _G, _L), (1, 0, 2))
    losses = pl.pallas_call(
        _loss_kernel,
        in_specs=[
            pl.BlockSpec((_G, _S, _L), lambda: (0, 0, 0)),
            pl.BlockSpec((_G, _S, _L), lambda: (0, 0, 0)),
        ],
        out_specs=pl.BlockSpec((_S, _L), lambda: (0, 0)),
        out_shape=jax.ShapeDtypeStruct((_S, _L), jnp.float32),
    )(lf, lb)
    return jnp.mean(losses[:, 0])


# speedup vs baseline: 8.6733x; 1.0210x over previous
import jax
import jax.numpy as jnp
from jax.experimental import pallas as pl
from jax.experimental.pallas import tpu as pltpu

_G = 2048
_S = 8
_L = 128
_LOGP = 18


def _loss_kernel(logits_ref, labels_ref, out_ref):
    x = logits_ref[...]
    lab = labels_ref[...]
    labf = lab.astype(jnp.float32)
    e = 1.0 - x * (2.0 * labf - 1.0)

    bits = pltpu.bitcast(e, jnp.int32)
    key = jnp.where(bits >= 0, bits, bits ^ jnp.int32(0x7FFFFFFF))
    key = (key & jnp.int32(-2)) | lab

    l_i = jax.lax.broadcasted_iota(jnp.int32, (1, 1, _L), 2)
    g_i0 = jax.lax.broadcasted_iota(jnp.int32, (_G, 1, 1), 0)

    def lane_bit(b):
        return ((l_i >> (b - 11)) & 1) == 1

    def flip_mask(ke):
        if ke <= 10:
            return ((g_i0 >> ke) & 1) - jnp.int32(1)
        if ke <= 17:
            return ((l_i >> (ke - 11)) & 1) - jnp.int32(1)
        return jnp.int32(-1)

    def low_merge(key, kbits):
        n = 1 << kbits
        w = key.reshape(_G // n, n, _S, _L)
        parts = [w[:, i] for i in range(n)]
        for jl in range(kbits - 1, -1, -1):
            d = 1 << jl
            for i in range(n):
                if not (i & d):
                    a, b = parts[i], parts[i + d]
                    parts[i] = jnp.minimum(a, b)
                    parts[i + d] = jnp.maximum(a, b)
        return jnp.concatenate(
            [p[:, None] for p in parts], axis=1).reshape(_G, _S, _L)

    n = 16
    w = key.reshape(_G // n, n, _S, _L)
    parts = [w[:, i] for i in range(n)]
    for ke in range(1, 4):
        for jl in range(ke - 1, -1, -1):
            d = 1 << jl
            for i in range(n):
                if not (i & d):
                    asc = ((i >> ke) & 1) == 1
                    a, b = parts[i], parts[i + d]
                    lo = jnp.minimum(a, b)
                    hi = jnp.maximum(a, b)
                    parts[i], parts[i + d] = (lo, hi) if asc else (hi, lo)
    key = jnp.concatenate(
        [p[:, None] for p in parts], axis=1).reshape(_G, _S, _L)

    _LOW = 5
    key = key ^ flip_mask(4)
    for ke in range(4, _LOGP + 1):
        for jl in range(ke - 1, 10, -1):
            d = 1 << (jl - 11)
            bit_u = lane_bit(jl)
            y = pltpu.roll(key, _L - d, 2)
            lo = jnp.minimum(key, y)
            hi = jnp.maximum(key, y)
            key = jnp.where(bit_u, pltpu.roll(hi, d, 2), lo)
        hi_top = min(ke - 1, 10)
        if hi_top >= _LOW:
            w = key.reshape(64, _G // 64, _S, _L)
            hparts = [w[i] for i in range(64)]
            for jl in range(hi_top, _LOW - 1, -1):
                pd = 1 << (jl - _LOW)
                for i in range(64):
                    if not (i & pd):
                        a, b = hparts[i], hparts[i + pd]
                        hparts[i] = jnp.minimum(a, b)
                        hparts[i + pd] = jnp.maximum(a, b)
            key = jnp.concatenate(
                [p[None] for p in hparts], axis=0).reshape(_G, _S, _L)
        key = low_merge(key, min(ke, _LOW))
        if ke < _LOGP:
            key = key ^ (flip_mask(ke) ^ flip_mask(ke + 1))
        else:
            key = key ^ flip_mask(ke)

    gt = (key & 1).astype(jnp.float32)
    ebits = jnp.where(key >= 0, key, key ^ jnp.int32(0x7FFFFFFF))
    relu_e = jnp.maximum(pltpu.bitcast(ebits, jnp.float32), 0.0)

    wg = gt.reshape(_G // 16, 16, _S, _L)
    qparts = [wg[:, i] for i in range(16)]
    for i in range(1, 16):
        qparts[i] = qparts[i] + qparts[i - 1]
    t1 = qparts[15]
    e1 = jnp.concatenate(
        [jnp.zeros((1, _S, _L), jnp.float32), t1[:-1]], axis=0)
    sh = 1
    while sh < _G // 16:
        e1 = e1 + jnp.concatenate(
            [jnp.zeros((sh, _S, _L), jnp.float32), e1[:-sh]], axis=0)
        sh *= 2
    c = jnp.concatenate(
        [(q + e1)[:, None] for q in qparts], axis=1).reshape(_G, _S, _L)
    t = c[_G - 1]
    ii = jax.lax.broadcasted_iota(jnp.int32, (_L, _L), 0)
    jj = jax.lax.broadcasted_iota(jnp.int32, (_L, _L), 1)
    lts = (ii < jj).astype(jnp.float32)
    p1 = jax.lax.dot(t, lts, precision=jax.lax.Precision.HIGHEST)
    cum_gt = c + p1[None]
    g_total = jnp.sum(t, axis=1)[None, :, None]

    g_i = jax.lax.broadcasted_iota(jnp.int32, (_G, 1, 1), 0)
    pos = (g_i + l_i * _G + 1).astype(jnp.float32)
    inter = g_total - cum_gt
    union = g_total + pos - cum_gt
    jacc = 1.0 - inter / union

    a = jnp.concatenate([jacc[_G - 1:], jacc[:-1]], axis=0)
    b2 = pltpu.roll(a, 1, 2)
    prev = jnp.where(g_i == 0, b2, a)
    prev = jnp.where((g_i == 0) & (l_i == 0), 0.0, prev)
    contrib = relu_e * (jacc - prev)
    part = jnp.sum(contrib, axis=0)
    ones_l = jnp.ones((_L, _L), jnp.float32)
    img_tot = jax.lax.dot(part, ones_l,
                          precision=jax.lax.Precision.HIGHEST)
    out_ref[...] = img_tot


def kernel(logits, labels):
    lf = jnp.transpose(logits.reshape(_S, _G, _L), (1, 0, 2))
    lb = jnp.transpose(labels.reshape(_S, _G, _L), (1, 0, 2))
    losses = pl.pallas_call(
        _loss_kernel,
        in_specs=[
            pl.BlockSpec((_G, _S, _L), lambda: (0, 0, 0)),
            pl.BlockSpec((_G, _S, _L), lambda: (0, 0, 0)),
        ],
        out_specs=pl.BlockSpec((_S, _L), lambda: (0, 0)),
        out_shape=jax.ShapeDtypeStruct((_S, _L), jnp.float32),
    )(lf, lb)
    return jnp.mean(losses[:, 0])
